# baseline (speedup 1.0000x reference)
import functools

import jax
import jax.numpy as jnp
import numpy as np
from jax import lax
from jax.experimental import pallas as pl
from jax.experimental.pallas import tpu as pltpu
from jax.sharding import Mesh, PartitionSpec as P

_N_HEADS = 4
_OUT_CH = 16
_C_TILE = 4096


def _body(et_ref, a_ref, wt_ref, ot_ref, ept_ref):
    @pl.when(pl.program_id(0) == 0)
    def _prologue():
        ept = jnp.concatenate(
            [
                lax.dot_general(
                    a_ref[h],
                    et_ref[h * _OUT_CH : (h + 1) * _OUT_CH, :],
                    (((1,), (0,)), ((), ())),
                    preferred_element_type=jnp.float32,
                )
                for h in range(_N_HEADS)
            ],
            axis=0,
        )
        ept_ref[...] = ept.astype(jnp.bfloat16)

    ot_ref[...] = lax.dot_general(
        wt_ref[...].astype(jnp.bfloat16),
        ept_ref[...],
        (((0,), (0,)), ((), ())),
        preferred_element_type=jnp.float32,
    )


def _local(et, a, wt):
    d, b = et.shape
    c_local = wt.shape[1]
    grid = (pl.cdiv(c_local, _C_TILE),)
    ot = pl.pallas_call(
        _body,
        grid=grid,
        in_specs=[
            pl.BlockSpec((d, b), lambda i: (0, 0)),
            pl.BlockSpec((_N_HEADS, _OUT_CH, _OUT_CH), lambda i: (0, 0, 0)),
            pl.BlockSpec((d, _C_TILE), lambda i: (0, i)),
        ],
        out_specs=pl.BlockSpec((_C_TILE, b), lambda i: (i, 0)),
        out_shape=jax.ShapeDtypeStruct((c_local, b), jnp.float32),
        scratch_shapes=[pltpu.VMEM((64, 1024), jnp.bfloat16)],
        compiler_params=pltpu.CompilerParams(
            dimension_semantics=("arbitrary",),
        ),
    )(et, a, wt)
    return ot


@functools.partial(jax.jit, static_argnames=())
def kernel(embeddings, emb_weight, attn_kernels):
    c = emb_weight.shape[0]
    et = embeddings.T
    wt = emb_weight.T
    devs = jax.devices()
    if len(devs) >= 2 and c % 2 == 0:
        mesh = Mesh(np.array(devs[:2]), ("x",))
        ot = jax.shard_map(
            _local,
            mesh=mesh,
            in_specs=(P(), P(), P(None, "x")),
            out_specs=P("x", None),
            check_vma=False,
        )(et, attn_kernels, wt)
    else:
        ot = _local(et, attn_kernels, wt)
    return ot.T


# speedup vs baseline: 2.4750x; 2.4750x over previous
import functools

import jax
import jax.numpy as jnp
from jax import lax
from jax.experimental import pallas as pl
from jax.experimental.pallas import tpu as pltpu

_N_HEADS = 4
_OUT_CH = 16
_C_TILE = 5120


def _body(et_ref, a_ref, wt_ref, ot_ref, ept_ref):
    @pl.when(pl.program_id(0) == 0)
    def _prologue():
        ept = jnp.concatenate(
            [
                lax.dot_general(
                    a_ref[h],
                    et_ref[h * _OUT_CH : (h + 1) * _OUT_CH, :],
                    (((1,), (0,)), ((), ())),
                    preferred_element_type=jnp.float32,
                )
                for h in range(_N_HEADS)
            ],
            axis=0,
        )
        ept_ref[...] = ept.astype(jnp.bfloat16)

    ot_ref[...] = lax.dot_general(
        wt_ref[...].astype(jnp.bfloat16),
        ept_ref[...],
        (((0,), (0,)), ((), ())),
        preferred_element_type=jnp.float32,
    )


@functools.partial(jax.jit, static_argnames=())
def kernel(embeddings, emb_weight, attn_kernels):
    b, d = embeddings.shape
    c = emb_weight.shape[0]
    et = embeddings.T
    wt = emb_weight.T
    grid = (pl.cdiv(c, _C_TILE),)
    ot = pl.pallas_call(
        _body,
        grid=grid,
        in_specs=[
            pl.BlockSpec((d, b), lambda i: (0, 0)),
            pl.BlockSpec((_N_HEADS, _OUT_CH, _OUT_CH), lambda i: (0, 0, 0)),
            pl.BlockSpec((d, _C_TILE), lambda i: (0, i)),
        ],
        out_specs=pl.BlockSpec((_C_TILE, b), lambda i: (i, 0)),
        out_shape=jax.ShapeDtypeStruct((c, b), jnp.float32),
        scratch_shapes=[pltpu.VMEM((64, 1024), jnp.bfloat16)],
        compiler_params=pltpu.CompilerParams(
            dimension_semantics=("arbitrary",),
        ),
    )(et, attn_kernels, wt)
    return ot.T


# speedup vs baseline: 2.4823x; 1.0029x over previous
import functools

import jax
import jax.numpy as jnp
from jax import lax
from jax.experimental import pallas as pl
from jax.experimental.pallas import tpu as pltpu

_N_HEADS = 4
_OUT_CH = 16
_C_TILE = 4096


def _body(et_ref, a_ref, wt_ref, ot_ref, ept_ref):
    @pl.when(pl.program_id(0) == 0)
    def _prologue():
        ept = jnp.concatenate(
            [
                lax.dot_general(
                    a_ref[h],
                    et_ref[h * _OUT_CH : (h + 1) * _OUT_CH, :],
                    (((1,), (0,)), ((), ())),
                    preferred_element_type=jnp.float32,
                )
                for h in range(_N_HEADS)
            ],
            axis=0,
        )
        ept_ref[...] = ept.astype(jnp.bfloat16)

    ot_ref[...] = lax.dot_general(
        wt_ref[...].astype(jnp.bfloat16),
        ept_ref[...],
        (((0,), (0,)), ((), ())),
        preferred_element_type=jnp.float32,
    )


@functools.partial(jax.jit, static_argnames=())
def kernel(embeddings, emb_weight, attn_kernels):
    b, d = embeddings.shape
    c = emb_weight.shape[0]
    et = embeddings.T
    wt = emb_weight.T
    grid = (pl.cdiv(c, _C_TILE),)
    ot = pl.pallas_call(
        _body,
        grid=grid,
        in_specs=[
            pl.BlockSpec((d, b), lambda i: (0, 0)),
            pl.BlockSpec((_N_HEADS, _OUT_CH, _OUT_CH), lambda i: (0, 0, 0)),
            pl.BlockSpec((d, _C_TILE), lambda i: (0, i)),
        ],
        out_specs=pl.BlockSpec((_C_TILE, b), lambda i: (i, 0)),
        out_shape=jax.ShapeDtypeStruct((c, b), jnp.float32),
        scratch_shapes=[pltpu.VMEM((64, 1024), jnp.bfloat16)],
        compiler_params=pltpu.CompilerParams(
            dimension_semantics=("arbitrary",),
        ),
    )(et, attn_kernels, wt)
    return ot.T
